# baseline (speedup 1.0000x reference)
import functools

import jax
import jax.numpy as jnp
from jax.experimental import pallas as pl

QK_DIM = 32
TOPK = 16
SCALE = QK_DIM ** (-0.5)
BATCH_BLOCK = 32


def _topk_body(q_ref, k_ref, w_ref, i_ref):
    q = q_ref[...] * SCALE
    k = k_ref[...]
    x = jax.lax.dot_general(
        k, q,
        dimension_numbers=(((2,), (2,)), ((0,), (0,))),
        preferred_element_type=jnp.float32,
    )
    p = x.shape[1]
    iota = jax.lax.broadcasted_iota(jnp.int32, x.shape, 1)

    vals = []
    for t in range(TOPK):
        m = jnp.max(x, axis=1)
        at_max = x == m[:, None, :]
        idx = jnp.min(jnp.where(at_max, iota, p), axis=1)
        vals.append(m)
        i_ref[:, t, :] = idx
        if t + 1 < TOPK:
            x = jnp.where(iota == idx[:, None, :], -jnp.inf, x)

    nums = [jnp.ones_like(vals[0])]
    den = nums[0]
    for t in range(1, TOPK):
        e = jnp.exp(vals[t] - vals[0])
        nums.append(e)
        den = den + e
    inv = 1.0 / den
    for t in range(TOPK):
        w_ref[:, t, :] = nums[t] * inv


@functools.partial(jax.jit, static_argnames=("interpret",))
def kernel(query, key, interpret=False):
    n, p, d = query.shape
    b = BATCH_BLOCK
    grid = (n // b,)
    w_t, i_t = pl.pallas_call(
        _topk_body,
        grid=grid,
        in_specs=[
            pl.BlockSpec((b, p, d), lambda i: (i, 0, 0)),
            pl.BlockSpec((b, p, d), lambda i: (i, 0, 0)),
        ],
        out_specs=[
            pl.BlockSpec((b, TOPK, p), lambda i: (i, 0, 0)),
            pl.BlockSpec((b, TOPK, p), lambda i: (i, 0, 0)),
        ],
        out_shape=[
            jax.ShapeDtypeStruct((n, TOPK, p), jnp.float32),
            jax.ShapeDtypeStruct((n, TOPK, p), jnp.int32),
        ],
        interpret=interpret,
    )(query, key)
    return jnp.transpose(w_t, (0, 2, 1)), jnp.transpose(i_t, (0, 2, 1))


# speedup vs baseline: 1.5119x; 1.5119x over previous
import functools

import jax
import jax.numpy as jnp
from jax.experimental import pallas as pl

QK_DIM = 32
TOPK = 16
SCALE = QK_DIM ** (-0.5)
BATCH_BLOCK = 16


def _topk_body(q_ref, k_ref, w_ref, i_ref):
    q = q_ref[...] * SCALE
    k = k_ref[...]
    x = jax.lax.dot_general(
        k, q,
        dimension_numbers=(((2,), (2,)), ((0,), (0,))),
        preferred_element_type=jnp.float32,
    )
    iota = jax.lax.broadcasted_iota(jnp.int32, x.shape, 1)

    def argmax_keys(x):
        val, idx = x, iota
        h = val.shape[1]
        while h > 1:
            h //= 2
            a_val, b_val = val[:, :h], val[:, h:]
            a_idx, b_idx = idx[:, :h], idx[:, h:]
            take_a = a_val >= b_val
            val = jnp.where(take_a, a_val, b_val)
            idx = jnp.where(take_a, a_idx, b_idx)
        return val[:, 0], idx[:, 0]

    vals = []
    for t in range(TOPK):
        m, idx = argmax_keys(x)
        vals.append(m)
        i_ref[:, t, :] = idx
        if t + 1 < TOPK:
            x = jnp.where(iota == idx[:, None, :], -jnp.inf, x)

    nums = [jnp.ones_like(vals[0])]
    den = nums[0]
    for t in range(1, TOPK):
        e = jnp.exp(vals[t] - vals[0])
        nums.append(e)
        den = den + e
    inv = 1.0 / den
    for t in range(TOPK):
        w_ref[:, t, :] = nums[t] * inv


@functools.partial(jax.jit, static_argnames=("interpret",))
def kernel(query, key, interpret=False):
    n, p, d = query.shape
    b = BATCH_BLOCK
    grid = (n // b,)
    w_t, i_t = pl.pallas_call(
        _topk_body,
        grid=grid,
        in_specs=[
            pl.BlockSpec((b, p, d), lambda i: (i, 0, 0)),
            pl.BlockSpec((b, p, d), lambda i: (i, 0, 0)),
        ],
        out_specs=[
            pl.BlockSpec((b, TOPK, p), lambda i: (i, 0, 0)),
            pl.BlockSpec((b, TOPK, p), lambda i: (i, 0, 0)),
        ],
        out_shape=[
            jax.ShapeDtypeStruct((n, TOPK, p), jnp.float32),
            jax.ShapeDtypeStruct((n, TOPK, p), jnp.int32),
        ],
        interpret=interpret,
    )(query, key)
    return jnp.transpose(w_t, (0, 2, 1)), jnp.transpose(i_t, (0, 2, 1))
